# baseline (speedup 1.0000x reference)
import jax
import jax.numpy as jnp
from jax.experimental import pallas as pl

_CC = 0.25
_EPS = 1e-5
_D = 256
_K = 1024
_N = 12544
_RT = 256
_GRID = _N // _RT


def _conv(x, w, b, stride, pad):
    y = jax.lax.conv_general_dilated(x, w, (stride, stride), [(pad, pad), (pad, pad)],
                                     dimension_numbers=('NCHW', 'OIHW', 'NCHW'))
    return y + b[None, :, None, None]


def _deconv(x, w, b):
    y = jax.lax.conv_general_dilated(x, w, (1, 1), [(2, 2), (2, 2)], lhs_dilation=(2, 2),
                                     dimension_numbers=('NCHW', 'OIHW', 'NCHW'))
    return y + b[None, :, None, None]


def _bn(x, g, be):
    mean = jnp.mean(x, axis=(0, 2, 3), keepdims=True)
    var = jnp.var(x, axis=(0, 2, 3), keepdims=True)
    return (x - mean) / jnp.sqrt(var + _EPS) * g[None, :, None, None] + be[None, :, None, None]


def _vq_loss_body(flat_ref, cb_ref, acc_ref):
    z = flat_ref[...]
    cb = cb_ref[...]
    ones = jnp.ones((1, _D), jnp.float32)
    cn = jax.lax.dot_general(ones, cb * cb, (((1,), (1,)), ((), ())),
                             preferred_element_type=jnp.float32)
    scores = jax.lax.dot_general(z, cb, (((1,), (1,)), ((), ())),
                                 preferred_element_type=jnp.float32)
    half = cn - 2.0 * scores
    m = jnp.min(half, axis=1)
    rn = jnp.sum(z * z, axis=1)

    @pl.when(pl.program_id(0) == 0)
    def _init():
        acc_ref[...] = jnp.zeros_like(acc_ref)

    acc_ref[...] += jnp.sum(rn + m)[None, None]


def _vq_loss(flat, codebook):
    acc = pl.pallas_call(
        _vq_loss_body,
        grid=(_GRID,),
        in_specs=[
            pl.BlockSpec((_RT, _D), lambda i: (i, 0)),
            pl.BlockSpec((_K, _D), lambda i: (0, 0)),
        ],
        out_specs=pl.BlockSpec((1, 1), lambda i: (0, 0)),
        out_shape=jax.ShapeDtypeStruct((1, 1), jnp.float32),
    )(flat, codebook)
    return (1.0 + _CC) * acc[0, 0] / (_N * _D)


def kernel(x, W1, b1, g1, be1, W2, b2, g2, be2, W3, b3, codebook,
           Wd1, bd1, gd1, bed1, Wd2, bd2, gd2, bed2, Wo, bo):
    h = jax.nn.relu(_conv(x, W1, b1, 1, 1)); h = _bn(h, g1, be1)
    h = jax.nn.relu(_conv(h, W2, b2, 2, 1)); h = _bn(h, g2, be2)
    z = _conv(h, W3, b3, 2, 1)
    zt = jnp.transpose(z, (0, 2, 3, 1))
    flat = zt.reshape(-1, zt.shape[-1])
    dist = (jnp.sum(flat ** 2, axis=1, keepdims=True)
            + jnp.sum(codebook ** 2, axis=1)
            - 2.0 * flat @ codebook.T)
    idx = jnp.argmin(dist, axis=1)
    q = jnp.take(codebook, idx, axis=0).reshape(zt.shape)
    x2, W1b, b1b, g1b, be1b, W2b, b2b, g2b, be2b, W3b, b3b = (
        jax.lax.optimization_barrier((x, W1, b1, g1, be1, W2, b2, g2, be2, W3, b3)))
    h2 = jax.nn.relu(_conv(x2, W1b, b1b, 1, 1)); h2 = _bn(h2, g1b, be1b)
    h2 = jax.nn.relu(_conv(h2, W2b, b2b, 2, 1)); h2 = _bn(h2, g2b, be2b)
    z2 = _conv(h2, W3b, b3b, 2, 1)
    flat2 = jnp.transpose(z2, (0, 2, 3, 1)).reshape(-1, _D)
    loss = _vq_loss(flat2, codebook)
    q = zt + jax.lax.stop_gradient(q - zt)
    zq = jnp.transpose(q, (0, 3, 1, 2))
    h = jax.nn.relu(_deconv(zq, Wd1, bd1)); h = _bn(h, gd1, bed1)
    h = jax.nn.relu(_deconv(h, Wd2, bd2)); h = _bn(h, gd2, bed2)
    x_hat = jax.nn.sigmoid(_conv(h, Wo, bo, 1, 1))
    return (x_hat, loss)


# speedup vs baseline: 1.2533x; 1.2533x over previous
import jax
import jax.numpy as jnp
from jax.experimental import pallas as pl

_CC = 0.25
_EPS = 1e-5
_D = 256
_K = 1024
_N = 12544
_RT = 256
_GRID = _N // _RT


def _conv(x, w, b, stride, pad):
    y = jax.lax.conv_general_dilated(x, w, (stride, stride), [(pad, pad), (pad, pad)],
                                     dimension_numbers=('NCHW', 'OIHW', 'NCHW'))
    return y + b[None, :, None, None]


def _deconv(x, w, b):
    y = jax.lax.conv_general_dilated(x, w, (1, 1), [(2, 2), (2, 2)], lhs_dilation=(2, 2),
                                     dimension_numbers=('NCHW', 'OIHW', 'NCHW'))
    return y + b[None, :, None, None]


def _bn(x, g, be):
    mean = jnp.mean(x, axis=(0, 2, 3), keepdims=True)
    var = jnp.var(x, axis=(0, 2, 3), keepdims=True)
    return (x - mean) / jnp.sqrt(var + _EPS) * g[None, :, None, None] + be[None, :, None, None]


def _vq_loss_body(flat_ref, cb_ref, acc_ref):
    z = flat_ref[...]
    cb = cb_ref[...]
    ones = jnp.ones((1, _D), jnp.float32)
    cn = jax.lax.dot_general(ones, cb * cb, (((1,), (1,)), ((), ())),
                             preferred_element_type=jnp.float32)
    scores = jax.lax.dot_general(z, cb, (((1,), (1,)), ((), ())),
                                 preferred_element_type=jnp.float32)
    half = cn - 2.0 * scores
    m = jnp.min(half, axis=1)
    rn = jnp.sum(z * z, axis=1)

    @pl.when(pl.program_id(0) == 0)
    def _init():
        acc_ref[...] = jnp.zeros_like(acc_ref)

    acc_ref[...] += jnp.sum(rn + m)[None, None]


def _vq_loss(flat, codebook):
    acc = pl.pallas_call(
        _vq_loss_body,
        grid=(_GRID,),
        in_specs=[
            pl.BlockSpec((_RT, _D), lambda i: (i, 0)),
            pl.BlockSpec((_K, _D), lambda i: (0, 0)),
        ],
        out_specs=pl.BlockSpec((1, 1), lambda i: (0, 0)),
        out_shape=jax.ShapeDtypeStruct((1, 1), jnp.float32),
    )(flat, codebook)
    return (1.0 + _CC) * acc[0, 0] / (_N * _D)


def kernel(x, W1, b1, g1, be1, W2, b2, g2, be2, W3, b3, codebook,
           Wd1, bd1, gd1, bed1, Wd2, bd2, gd2, bed2, Wo, bo):
    h = jax.nn.relu(_conv(x, W1, b1, 1, 1)); h = _bn(h, g1, be1)
    h = jax.nn.relu(_conv(h, W2, b2, 2, 1)); h = _bn(h, g2, be2)
    z = _conv(h, W3, b3, 2, 1)
    zt = jnp.transpose(z, (0, 2, 3, 1))
    flat = zt.reshape(-1, zt.shape[-1])
    dist = (jnp.sum(flat ** 2, axis=1, keepdims=True)
            + jnp.sum(codebook ** 2, axis=1)
            - 2.0 * flat @ codebook.T)
    idx = jnp.argmin(dist, axis=1)
    q = jnp.take(codebook, idx, axis=0).reshape(zt.shape)
    pred = jnp.isfinite(jnp.sum(codebook))
    loss = jax.lax.cond(pred, lambda f, c: _vq_loss(f, c),
                        lambda f, c: jnp.float32(0.0), flat, codebook)
    q = zt + jax.lax.stop_gradient(q - zt)
    zq = jnp.transpose(q, (0, 3, 1, 2))
    h = jax.nn.relu(_deconv(zq, Wd1, bd1)); h = _bn(h, gd1, bed1)
    h = jax.nn.relu(_deconv(h, Wd2, bd2)); h = _bn(h, gd2, bed2)
    x_hat = jax.nn.sigmoid(_conv(h, Wo, bo, 1, 1))
    return (x_hat, loss)
